# baseline (device time: 173491 ns/iter reference)
import functools

import jax
import jax.numpy as jnp
from jax import lax
from jax.experimental import pallas as pl
from jax.experimental.pallas import tpu as pltpu

N_DEV = 8
K_SLOT = 4
N_STREAM = 4
CREDIT_FIRST_GATED_SEND = K_SLOT - 1
CREDIT_LAST_SIGNAL_STEP = (N_DEV - 2) - (K_SLOT - 1)


def kernel(x, w_mat):
    m, k_shard = x.shape
    _, n = w_mat.shape
    m_chunk = m // N_DEV
    nq = n // 4

    def body(x_ref, w_ref, out_ref,
             bufs, send_sems, recv_sems, credit_sems):
        my = lax.axis_index("i")
        left = (my - 1) % N_DEV
        right = (my + 1) % N_DEV

        barrier_sem = pltpu.get_barrier_semaphore()
        for nbr in (left, right):
            pl.semaphore_signal(
                barrier_sem, inc=1,
                device_id=(nbr,), device_id_type=pl.DeviceIdType.MESH,
            )

        def fchunk(c):
            rows = x_ref[pl.ds(c * m_chunk, m_chunk), :]
            return jnp.dot(rows, w_ref[:, : 2 * nq],
                           preferred_element_type=jnp.float32)

        def bchunk(c):
            rows = x_ref[pl.ds(c * m_chunk, m_chunk), :]
            return jnp.dot(rows, w_ref[:, 2 * nq:],
                           preferred_element_type=jnp.float32)

        STREAMS = (
            dict(i=0, dst=1, peer=-1, qcol=0),
            dict(i=2, dst=-1, peer=1, qcol=2),
            dict(i=1, dst=1, peer=-1, qcol=1),
            dict(i=3, dst=-1, peer=1, qcol=3),
        )

        def send_chunk(st):
            def _f(s):
                if st["dst"] == 1:
                    return (my + N_DEV - 1 - s) % N_DEV
                return (my + 1 + s) % N_DEV
            return _f

        def make_rdma(st, s):
            i = st["i"]
            dev = right if st["dst"] == 1 else left
            return pltpu.make_async_remote_copy(
                src_ref=bufs[i].at[s % K_SLOT],
                dst_ref=bufs[i].at[(s + 1) % K_SLOT],
                send_sem=send_sems.at[i, s % K_SLOT],
                recv_sem=recv_sems.at[i, (s + 1) % K_SLOT],
                device_id=(dev,),
                device_id_type=pl.DeviceIdType.MESH,
            )

        cf0 = (my + N_DEV - 1) % N_DEV
        cb0 = (my + 1) % N_DEV
        rows_f0 = x_ref[pl.ds(cf0 * m_chunk, m_chunk), :]
        rows_b0 = x_ref[pl.ds(cb0 * m_chunk, m_chunk), :]
        cur = {}
        first = True
        for st in STREAMS:
            i = st["i"]
            q = st["qcol"]
            rows = rows_f0 if st["dst"] == 1 else rows_b0
            bufs[i][0, :, :] = jnp.dot(
                rows, w_ref[:, q * nq:(q + 1) * nq],
                preferred_element_type=jnp.float32,
            )
            if first:
                pl.semaphore_wait(barrier_sem, 2)
                first = False
            cur[i] = make_rdma(st, 0)
            cur[i].start()

        for s in range(N_DEV - 1):
            slot = (s + 1) % K_SLOT
            tf = fchunk((my + N_DEV - 2 - s) % N_DEV)
            tb = bchunk((my + 2 + s) % N_DEV)
            temps = {0: tf[:, :nq], 1: tf[:, nq:], 2: tb[:, :nq], 3: tb[:, nq:]}

            for st in STREAMS:
                i = st["i"]
                cur[i].wait()
                if s <= CREDIT_LAST_SIGNAL_STEP:
                    pl.semaphore_signal(
                        credit_sems.at[i], inc=1,
                        device_id=((my + st["peer"]) % N_DEV,),
                        device_id_type=pl.DeviceIdType.MESH,
                    )
                acc = bufs[i][slot, :, :] + temps[i]
                if s < N_DEV - 2:
                    bufs[i][slot, :, :] = acc
                    if s + 1 >= CREDIT_FIRST_GATED_SEND:
                        pl.semaphore_wait(credit_sems.at[i], 1)
                    cur[i] = make_rdma(st, s + 1)
                    cur[i].start()
                else:
                    q = st["qcol"]
                    out_ref[:, q * nq:(q + 1) * nq] = acc * (
                        1.0 / (1.0 + jnp.exp(-acc))
                    )

        @functools.partial(
            pl.run_scoped, second_barrier=pltpu.SemaphoreType.REGULAR
        )
        def _(second_barrier):
            for nbr in (left, right):
                pl.semaphore_signal(
                    second_barrier, inc=1,
                    device_id=(nbr,), device_id_type=pl.DeviceIdType.MESH,
                )
            pl.semaphore_wait(second_barrier, 2)

    def wrapped_body(x_ref, w_ref, out_ref,
                     buf0, buf1, buf2, buf3,
                     send_sems, recv_sems, credit_sems):
        body(x_ref, w_ref, out_ref, [buf0, buf1, buf2, buf3],
             send_sems, recv_sems, credit_sems)

    return pl.pallas_call(
        wrapped_body,
        out_shape=jax.ShapeDtypeStruct((m_chunk, n), jnp.float32),
        in_specs=[
            pl.BlockSpec(memory_space=pltpu.VMEM),
            pl.BlockSpec(memory_space=pltpu.VMEM),
        ],
        out_specs=pl.BlockSpec(memory_space=pltpu.VMEM),
        scratch_shapes=[
            pltpu.VMEM((K_SLOT, m_chunk, nq), jnp.float32),
            pltpu.VMEM((K_SLOT, m_chunk, nq), jnp.float32),
            pltpu.VMEM((K_SLOT, m_chunk, nq), jnp.float32),
            pltpu.VMEM((K_SLOT, m_chunk, nq), jnp.float32),
            pltpu.SemaphoreType.DMA((N_STREAM, K_SLOT)),
            pltpu.SemaphoreType.DMA((N_STREAM, K_SLOT)),
            pltpu.SemaphoreType.REGULAR((N_STREAM,)),
        ],
        compiler_params=pltpu.CompilerParams(collective_id=0),
    )(x, w_mat)


# device time: 150319 ns/iter; 1.1542x vs baseline; 1.1542x over previous
import functools

import jax
import jax.numpy as jnp
from jax import lax
from jax.experimental import pallas as pl
from jax.experimental.pallas import tpu as pltpu

N_DEV = 8
N_PLANE = 4
N_STEP = 6
SEND_SLOT = (0, 1, 2, 4, 0, 1)
RECV_SLOT = (1, 2, 3, 0, 1, 2)
Z_SLOT = 3
I_SLOT = 4


def kernel(x, w_mat):
    m, k_shard = x.shape
    _, n = w_mat.shape
    m_chunk = m // N_DEV
    nq = n // 4

    def body(x_ref, w_ref, out_ref,
             buf0, buf1, buf2, buf3, zrecv_ref,
             send_sems, recv_sems, zsend_sems, zrecv_sems, credit_sems):
        bufs = [buf0, buf1, buf2, buf3]
        my = lax.axis_index("i")
        zid = my // N_PLANE
        p = my % N_PLANE
        right = zid * N_PLANE + (p + 1) % N_PLANE
        left = zid * N_PLANE + (p + 3) % N_PLANE
        partner = (my + N_PLANE) % N_DEV
        sb = N_PLANE * (1 - zid)
        kb = N_PLANE * zid

        barrier_sem = pltpu.get_barrier_semaphore()
        for nbr in (left, right, partner):
            pl.semaphore_signal(
                barrier_sem, inc=1,
                device_id=(nbr,), device_id_type=pl.DeviceIdType.MESH,
            )

        def dot_cols(c, c0, c1):
            rows = x_ref[pl.ds(c * m_chunk, m_chunk), :]
            return jnp.dot(rows, w_ref[:, c0:c1],
                           preferred_element_type=jnp.float32)

        def f_send_chunk(B, s):
            return B + (p + 3 - s) % N_PLANE

        def f_recv_chunk(B, s):
            return B + (p + 2 - s) % N_PLANE

        def b_send_chunk(B, s):
            return B + (p + 1 + s) % N_PLANE

        def b_recv_chunk(B, s):
            return B + (p + 2 + s) % N_PLANE

        STREAMS = (
            dict(i=0, dirn=1, qcol=0),
            dict(i=2, dirn=-1, qcol=2),
            dict(i=1, dirn=1, qcol=1),
            dict(i=3, dirn=-1, qcol=3),
        )

        def st_chunk(st, B, s, recv):
            if st["dirn"] == 1:
                return f_recv_chunk(B, s) if recv else f_send_chunk(B, s)
            return b_recv_chunk(B, s) if recv else b_send_chunk(B, s)

        def make_rdma(st, t):
            i = st["i"]
            dev = right if st["dirn"] == 1 else left
            return pltpu.make_async_remote_copy(
                src_ref=bufs[i].at[SEND_SLOT[t]],
                dst_ref=bufs[i].at[RECV_SLOT[t]],
                send_sem=send_sems.at[i, t % 3],
                recv_sem=recv_sems.at[i, t % 3],
                device_id=(dev,),
                device_id_type=pl.DeviceIdType.MESH,
            )

        for st in STREAMS:
            i, q = st["i"], st["qcol"]
            c = st_chunk(st, sb, 0, recv=False)
            bufs[i][0, :, :] = dot_cols(c, q * nq, (q + 1) * nq)

        pl.semaphore_wait(barrier_sem, 3)

        cur = {}
        for st in STREAMS:
            cur[st["i"]] = make_rdma(st, 0)
            cur[st["i"]].start()

        zcopies = {}
        for t in range(N_STEP):
            B = sb if t < 3 else kb
            s = t % 3
            tf = dot_cols(f_recv_chunk(B, s), 0, 2 * nq)
            tb = dot_cols(b_recv_chunk(B, s), 2 * nq, n)
            temps = {0: tf[:, :nq], 1: tf[:, nq:],
                     2: tb[:, :nq], 3: tb[:, nq:]}
            if t == 1:
                for st in STREAMS:
                    i, q = st["i"], st["qcol"]
                    c = st_chunk(st, kb, 0, recv=False)
                    bufs[i][I_SLOT, :, :] = dot_cols(c, q * nq, (q + 1) * nq)

            for st in STREAMS:
                i, q = st["i"], st["qcol"]
                cur[i].wait()
                if t <= 2:
                    upstream = left if st["dirn"] == 1 else right
                    pl.semaphore_signal(
                        credit_sems.at[i], inc=1,
                        device_id=(upstream,),
                        device_id_type=pl.DeviceIdType.MESH,
                    )
                acc = bufs[i][RECV_SLOT[t], :, :] + temps[i]
                if t == N_STEP - 1:
                    zcopies[i].wait()
                    tot = acc + zrecv_ref[i, :, :]
                    out_ref[:, q * nq:(q + 1) * nq] = tot * (
                        1.0 / (1.0 + jnp.exp(-tot))
                    )
                else:
                    bufs[i][RECV_SLOT[t], :, :] = acc
                    if t == 2:
                        zc = pltpu.make_async_remote_copy(
                            src_ref=bufs[i].at[Z_SLOT],
                            dst_ref=zrecv_ref.at[i],
                            send_sem=zsend_sems.at[i],
                            recv_sem=zrecv_sems.at[i],
                            device_id=(partner,),
                            device_id_type=pl.DeviceIdType.MESH,
                        )
                        zc.start()
                        zcopies[i] = zc
                    if t + 1 >= 3:
                        pl.semaphore_wait(credit_sems.at[i], 1)
                    cur[i] = make_rdma(st, t + 1)
                    cur[i].start()

        @functools.partial(
            pl.run_scoped, second_barrier=pltpu.SemaphoreType.REGULAR
        )
        def _(second_barrier):
            for nbr in (left, right, partner):
                pl.semaphore_signal(
                    second_barrier, inc=1,
                    device_id=(nbr,), device_id_type=pl.DeviceIdType.MESH,
                )
            pl.semaphore_wait(second_barrier, 3)

    return pl.pallas_call(
        body,
        out_shape=jax.ShapeDtypeStruct((m_chunk, n), jnp.float32),
        in_specs=[
            pl.BlockSpec(memory_space=pltpu.VMEM),
            pl.BlockSpec(memory_space=pltpu.VMEM),
        ],
        out_specs=pl.BlockSpec(memory_space=pltpu.VMEM),
        scratch_shapes=[
            pltpu.VMEM((5, m_chunk, nq), jnp.float32),
            pltpu.VMEM((5, m_chunk, nq), jnp.float32),
            pltpu.VMEM((5, m_chunk, nq), jnp.float32),
            pltpu.VMEM((5, m_chunk, nq), jnp.float32),
            pltpu.VMEM((4, m_chunk, nq), jnp.float32),
            pltpu.SemaphoreType.DMA((4, 3)),
            pltpu.SemaphoreType.DMA((4, 3)),
            pltpu.SemaphoreType.DMA((4,)),
            pltpu.SemaphoreType.DMA((4,)),
            pltpu.SemaphoreType.REGULAR((4,)),
        ],
        compiler_params=pltpu.CompilerParams(collective_id=0),
    )(x, w_mat)


# device time: 133827 ns/iter; 1.2964x vs baseline; 1.1232x over previous
import functools

import jax
import jax.numpy as jnp
from jax import lax
from jax.experimental import pallas as pl
from jax.experimental.pallas import tpu as pltpu

N_DEV = 8
N_PLANE = 4
N_STEP = 6
SEND_SLOT = (0, 1, 2, 4, 0, 1)
RECV_SLOT = (1, 2, 3, 0, 1, 2)
Z_SLOT = 3
I_SLOT = 4
SEND2 = (0, 1, 2)
RECV2 = (1, 2, 0)

NP_COLS = 1536
NQP = 384
NZ0 = 1536
NQZ = 256


def kernel(x, w_mat):
    m, k_shard = x.shape
    _, n = w_mat.shape
    m_chunk = m // N_DEV

    def body(x_ref, w_ref, out_ref,
             buf0, buf1, buf2, buf3, zrecvP_ref,
             zxbuf_ref, zrbuf_ref, z2buf0, z2buf1,
             send_sems, recv_sems, zsendP_sems, zrecvP_sems, creditP_sems,
             zx_send_sems, zx_recv_sems, z2_send_sems, z2_recv_sems,
             credit2_sems):
        bufs = [buf0, buf1, buf2, buf3]
        z2bufs = [z2buf0, z2buf1]
        my = lax.axis_index("i")
        zid = my // N_PLANE
        p = my % N_PLANE
        right = zid * N_PLANE + (p + 1) % N_PLANE
        left = zid * N_PLANE + (p + 3) % N_PLANE
        partner = (my + N_PLANE) % N_DEV
        sb = N_PLANE * (1 - zid)
        kb = N_PLANE * zid

        barrier_sem = pltpu.get_barrier_semaphore()
        for nbr in (left, right, partner):
            pl.semaphore_signal(
                barrier_sem, inc=1,
                device_id=(nbr,), device_id_type=pl.DeviceIdType.MESH,
            )

        def dot_cols(c, c0, c1):
            rows = x_ref[pl.ds(c * m_chunk, m_chunk), :]
            return jnp.dot(rows, w_ref[:, c0:c1],
                           preferred_element_type=jnp.float32)

        def f_send_chunk(B, s):
            return B + (p + 3 - s) % N_PLANE

        def f_recv_chunk(B, s):
            return B + (p + 2 - s) % N_PLANE

        def b_send_chunk(B, s):
            return B + (p + 1 + s) % N_PLANE

        def b_recv_chunk(B, s):
            return B + (p + 2 + s) % N_PLANE

        STREAMS = (
            dict(i=0, dirn=1, qcol=0),
            dict(i=2, dirn=-1, qcol=2),
            dict(i=1, dirn=1, qcol=1),
            dict(i=3, dirn=-1, qcol=3),
        )

        def st_chunk(st, B, s, recv):
            if st["dirn"] == 1:
                return f_recv_chunk(B, s) if recv else f_send_chunk(B, s)
            return b_recv_chunk(B, s) if recv else b_send_chunk(B, s)

        def make_rdma(st, t):
            i = st["i"]
            dev = right if st["dirn"] == 1 else left
            return pltpu.make_async_remote_copy(
                src_ref=bufs[i].at[SEND_SLOT[t]],
                dst_ref=bufs[i].at[RECV_SLOT[t]],
                send_sem=send_sems.at[i, t % 3],
                recv_sem=recv_sems.at[i, t % 3],
                device_id=(dev,),
                device_id_type=pl.DeviceIdType.MESH,
            )

        J = ((p + 3) % N_PLANE, (p + 1) % N_PLANE, (p + 2) % N_PLANE, p)
        zx_descs = {}

        def zx_start(r):
            zxbuf_ref[r, :, :] = dot_cols(sb + J[r], NZ0, n)
            d = pltpu.make_async_remote_copy(
                src_ref=zxbuf_ref.at[r],
                dst_ref=zrbuf_ref.at[r],
                send_sem=zx_send_sems.at[r],
                recv_sem=zx_recv_sems.at[r],
                device_id=(partner,),
                device_id_type=pl.DeviceIdType.MESH,
            )
            d.start()
            zx_descs[r] = d

        zx_waited = set()

        def zx_ensure(r):
            if r not in zx_waited:
                zx_descs[r].wait_recv()
                zx_waited.add(r)

        STREAMS2 = (
            dict(i=0, dirn=1, c0=NZ0, r_send0=0, r_recv=(2, 1, 3)),
            dict(i=1, dirn=-1, c0=NZ0 + NQZ, r_send0=1, r_recv=(2, 0, 3)),
        )

        def zred(st2, c, r):
            zx_ensure(r)
            off = st2["c0"] - NZ0
            return (
                dot_cols(c, st2["c0"], st2["c0"] + NQZ)
                + zrbuf_ref[r, :, off:off + NQZ]
            )

        def make_rdma2(st2, s):
            i = st2["i"]
            dev = right if st2["dirn"] == 1 else left
            return pltpu.make_async_remote_copy(
                src_ref=z2bufs[i].at[SEND2[s]],
                dst_ref=z2bufs[i].at[RECV2[s]],
                send_sem=z2_send_sems.at[i, s],
                recv_sem=z2_recv_sems.at[i, s],
                device_id=(dev,),
                device_id_type=pl.DeviceIdType.MESH,
            )

        def st2_chunk(st2, s, recv):
            if st2["dirn"] == 1:
                return f_recv_chunk(kb, s) if recv else f_send_chunk(kb, s)
            return b_recv_chunk(kb, s) if recv else b_send_chunk(kb, s)

        for st in STREAMS:
            i, q = st["i"], st["qcol"]
            c = st_chunk(st, sb, 0, recv=False)
            bufs[i][0, :, :] = dot_cols(c, q * NQP, (q + 1) * NQP)

        pl.semaphore_wait(barrier_sem, 3)

        cur = {}
        for st in STREAMS:
            cur[st["i"]] = make_rdma(st, 0)
            cur[st["i"]].start()
        for r in range(4):
            zx_start(r)

        cur2 = {}
        zcopies = {}
        for t in range(N_STEP):
            B = sb if t < 3 else kb
            s = t % 3
            tf = dot_cols(f_recv_chunk(B, s), 0, 2 * NQP)
            tb = dot_cols(b_recv_chunk(B, s), 2 * NQP, 4 * NQP)
            temps = {0: tf[:, :NQP], 1: tf[:, NQP:],
                     2: tb[:, :NQP], 3: tb[:, NQP:]}
            if t == 1:
                for st in STREAMS:
                    i, q = st["i"], st["qcol"]
                    c = st_chunk(st, kb, 0, recv=False)
                    bufs[i][I_SLOT, :, :] = dot_cols(c, q * NQP, (q + 1) * NQP)

            for st in STREAMS:
                i, q = st["i"], st["qcol"]
                cur[i].wait()
                if t <= 2:
                    upstream = left if st["dirn"] == 1 else right
                    pl.semaphore_signal(
                        creditP_sems.at[i], inc=1,
                        device_id=(upstream,),
                        device_id_type=pl.DeviceIdType.MESH,
                    )
                acc = bufs[i][RECV_SLOT[t], :, :] + temps[i]
                if t == N_STEP - 1:
                    zcopies[i].wait()
                    tot = acc + zrecvP_ref[i, :, :]
                    out_ref[:, q * NQP:(q + 1) * NQP] = tot * (
                        1.0 / (1.0 + jnp.exp(-tot))
                    )
                else:
                    bufs[i][RECV_SLOT[t], :, :] = acc
                    if t == 2:
                        zc = pltpu.make_async_remote_copy(
                            src_ref=bufs[i].at[Z_SLOT],
                            dst_ref=zrecvP_ref.at[i],
                            send_sem=zsendP_sems.at[i],
                            recv_sem=zrecvP_sems.at[i],
                            device_id=(partner,),
                            device_id_type=pl.DeviceIdType.MESH,
                        )
                        zc.start()
                        zcopies[i] = zc
                    if t + 1 >= 3:
                        pl.semaphore_wait(creditP_sems.at[i], 1)
                    cur[i] = make_rdma(st, t + 1)
                    cur[i].start()

            if t == 1:
                for st2 in STREAMS2:
                    i = st2["i"]
                    z2bufs[i][0, :, :] = zred(
                        st2, st2_chunk(st2, 0, recv=False), st2["r_send0"]
                    )
                    cur2[i] = make_rdma2(st2, 0)
                    cur2[i].start()
            elif t in (3, 4):
                s2 = t - 3
                for st2 in STREAMS2:
                    i = st2["i"]
                    cur2[i].wait()
                    if s2 == 0:
                        upstream = left if st2["dirn"] == 1 else right
                        pl.semaphore_signal(
                            credit2_sems.at[i], inc=1,
                            device_id=(upstream,),
                            device_id_type=pl.DeviceIdType.MESH,
                        )
                    acc2 = z2bufs[i][RECV2[s2], :, :] + zred(
                        st2, st2_chunk(st2, s2, recv=True), st2["r_recv"][s2]
                    )
                    z2bufs[i][RECV2[s2], :, :] = acc2
                    if s2 + 1 == 2:
                        pl.semaphore_wait(credit2_sems.at[i], 1)
                    cur2[i] = make_rdma2(st2, s2 + 1)
                    cur2[i].start()
            elif t == 5:
                for st2 in STREAMS2:
                    i = st2["i"]
                    cur2[i].wait()
                    tot2 = z2bufs[i][RECV2[2], :, :] + zred(
                        st2, st2_chunk(st2, 2, recv=True), st2["r_recv"][2]
                    )
                    c0 = st2["c0"]
                    out_ref[:, c0:c0 + NQZ] = tot2 * (
                        1.0 / (1.0 + jnp.exp(-tot2))
                    )
                for r in range(4):
                    zx_descs[r].wait_send()

        @functools.partial(
            pl.run_scoped, second_barrier=pltpu.SemaphoreType.REGULAR
        )
        def _(second_barrier):
            for nbr in (left, right, partner):
                pl.semaphore_signal(
                    second_barrier, inc=1,
                    device_id=(nbr,), device_id_type=pl.DeviceIdType.MESH,
                )
            pl.semaphore_wait(second_barrier, 3)

    return pl.pallas_call(
        body,
        out_shape=jax.ShapeDtypeStruct((m_chunk, n), jnp.float32),
        in_specs=[
            pl.BlockSpec(memory_space=pltpu.VMEM),
            pl.BlockSpec(memory_space=pltpu.VMEM),
        ],
        out_specs=pl.BlockSpec(memory_space=pltpu.VMEM),
        scratch_shapes=[
            pltpu.VMEM((5, m_chunk, NQP), jnp.float32),
            pltpu.VMEM((5, m_chunk, NQP), jnp.float32),
            pltpu.VMEM((5, m_chunk, NQP), jnp.float32),
            pltpu.VMEM((5, m_chunk, NQP), jnp.float32),
            pltpu.VMEM((4, m_chunk, NQP), jnp.float32),
            pltpu.VMEM((4, m_chunk, n - NZ0), jnp.float32),
            pltpu.VMEM((4, m_chunk, n - NZ0), jnp.float32),
            pltpu.VMEM((3, m_chunk, NQZ), jnp.float32),
            pltpu.VMEM((3, m_chunk, NQZ), jnp.float32),
            pltpu.SemaphoreType.DMA((4, 3)),
            pltpu.SemaphoreType.DMA((4, 3)),
            pltpu.SemaphoreType.DMA((4,)),
            pltpu.SemaphoreType.DMA((4,)),
            pltpu.SemaphoreType.REGULAR((4,)),
            pltpu.SemaphoreType.DMA((4,)),
            pltpu.SemaphoreType.DMA((4,)),
            pltpu.SemaphoreType.DMA((2, 3)),
            pltpu.SemaphoreType.DMA((2, 3)),
            pltpu.SemaphoreType.REGULAR((2,)),
        ],
        compiler_params=pltpu.CompilerParams(collective_id=0),
    )(x, w_mat)
